# Initial kernel scaffold; baseline (speedup 1.0000x reference)
import jax
import jax.numpy as jnp
from jax.experimental import pallas as pl


def kernel(x, edge_index, attr, batch, params):
    raise NotImplementedError("write your pallas kernel here")


# speedup vs baseline: 7.5705x; 7.5705x over previous
import functools

import jax
import jax.numpy as jnp
from jax import lax
from jax.experimental import pallas as pl
from jax.experimental.pallas import tpu as pltpu
from jax.experimental.pallas import tpu_sc as plsc

H = 4
C = 32
D = H * C
F = 16
N = 10000
E = 320000
G = 16
CLS = 10
QQW = D + H * F
ROWW = D + H * F

NC, NS = 2, 16
NW = NC * NS
EPW = E // NW
B = 80
CH = EPW // B
STRIPE = N // NS
SCALE = 1.0 / (C ** 0.5)

ROWBLK = 2000
GRID = N // ROWBLK

_mesh = lambda: plsc.VectorSubcoreMesh(core_axis_name="c", subcore_axis_name="s")


def _iota16():
    return lax.iota(jnp.int32, 16)


def _full16(v):
    return jnp.full((16,), v, jnp.int32)


def _sc_alpha_body(qq_hbm, k_hbm, attr_hbm, src_hbm, dst_hbm, ex_hbm,
                   srcb, dstb, krows, qrows, attrb, exb, sem1, sem2):
    wid = lax.axis_index("c") * NS + lax.axis_index("s")

    def chunk(i, _):
        base = wid * EPW + i * B
        pltpu.sync_copy(src_hbm.at[pl.ds(base, B)], srcb)
        pltpu.sync_copy(dst_hbm.at[pl.ds(base, B)], dstb)
        cp1 = pltpu.async_copy(k_hbm.at[srcb], krows, sem1)
        cp2 = pltpu.async_copy(qq_hbm.at[dstb], qrows, sem2)
        pltpu.sync_copy(attr_hbm.at[pl.ds(base, B)], attrb)
        cp1.wait()
        cp2.wait()

        def group(g, _):
            elane = g * 16 + _iota16()
            attrv = [plsc.load_gather(attrb, [elane, _full16(f)])
                     for f in range(F)]
            for h in range(H):
                acc = jnp.zeros((16,), jnp.float32)
                for c in range(C):
                    col = _full16(h * C + c)
                    qv = plsc.load_gather(qrows, [elane, col])
                    kv = plsc.load_gather(krows, [elane, col])
                    acc = acc + qv * kv
                for f in range(F):
                    qwv = plsc.load_gather(qrows, [elane, _full16(D + h * F + f)])
                    acc = acc + attrv[f] * qwv
                exv = jnp.exp(acc * SCALE)
                plsc.store_scatter(exb, [elane, _full16(h)], exv)
            return _

        lax.fori_loop(0, B // 16, group, None)
        pltpu.sync_copy(exb, ex_hbm.at[pl.ds(base, B)])
        return _

    lax.fori_loop(0, CH, chunk, None)


def _sc_alpha(qq, k, attr, src, dst):
    return pl.kernel(
        _sc_alpha_body,
        jax.ShapeDtypeStruct((E, H), jnp.float32),
        mesh=_mesh(),
        compiler_params=pltpu.CompilerParams(use_tc_tiling_on_sc=False, needs_layout_passes=False),
        scratch_types=[
            pltpu.VMEM((B,), jnp.int32),
            pltpu.VMEM((B,), jnp.int32),
            pltpu.VMEM((B, D), jnp.float32),
            pltpu.VMEM((B, QQW), jnp.float32),
            pltpu.VMEM((B, F), jnp.float32),
            pltpu.VMEM((B, H), jnp.float32),
            pltpu.SemaphoreType.DMA,
            pltpu.SemaphoreType.DMA,
        ],
    )(qq, k, attr, src, dst)


AW1 = 128
AW2 = 80


def _sc_agg_body(aw, with_attr, v_hbm, attr_hbm, ex_hbm, src_hbm, dst_hbm,
                 acc_hbm, srcb, dstb, vrows, attrb, exb, scal, acc_s, sem1):
    cid = lax.axis_index("c")
    sid = lax.axis_index("s")
    wid = cid * NS + sid

    def zrow(r, _):
        for kk in range(aw // 16):
            scal[r, pl.ds(kk * 16, 16)] = jnp.zeros((16,), jnp.float32)
        return _

    lax.fori_loop(0, B, zrow, None)

    soff = sid * STRIPE
    for off in range(0, STRIPE - B + 1, B):
        pltpu.sync_copy(scal.at[pl.ds(0, B)], acc_s.at[pl.ds(soff + off, B)])
    rem = STRIPE % B
    if rem:
        pltpu.sync_copy(scal.at[pl.ds(0, rem)],
                        acc_s.at[pl.ds(soff + STRIPE - rem, rem)])
    plsc.subcore_barrier()

    def chunk(i, _):
        base = wid * EPW + i * B
        pltpu.sync_copy(src_hbm.at[pl.ds(base, B)], srcb)
        pltpu.sync_copy(dst_hbm.at[pl.ds(base, B)], dstb)
        cp1 = pltpu.async_copy(v_hbm.at[srcb], vrows, sem1)
        if with_attr:
            pltpu.sync_copy(attr_hbm.at[pl.ds(base, B)], attrb)
        pltpu.sync_copy(ex_hbm.at[pl.ds(base, B)], exb)
        cp1.wait()

        def group(g, _):
            elane = g * 16 + _iota16()
            exv = [plsc.load_gather(exb, [elane, _full16(h)]) for h in range(H)]
            for c in range(D // 2):
                col = _full16(c)
                vv = plsc.load_gather(vrows, [elane, col])
                plsc.store_scatter(scal, [elane, col],
                                   vv * exv[c // C if aw == AW1 else 2 + c // C])
            if with_attr:
                attrv = [plsc.load_gather(attrb, [elane, _full16(f)])
                         for f in range(F)]
                for h in range(H):
                    for f in range(F):
                        plsc.store_scatter(
                            scal, [elane, _full16(D // 2 + h * F + f)],
                            attrv[f] * exv[h])
            else:
                for h in range(H):
                    plsc.store_scatter(scal, [elane, _full16(D // 2 + h)],
                                       exv[h])
            return _

        lax.fori_loop(0, B // 16, group, None)
        pltpu.sync_copy(scal, acc_s.at[dstb], add=True)
        return _

    lax.fori_loop(0, CH, chunk, None)
    plsc.subcore_barrier()

    hoff = cid * N + sid * STRIPE
    pltpu.sync_copy(acc_s.at[pl.ds(soff, STRIPE)], acc_hbm.at[pl.ds(hoff, STRIPE)])


def _sc_agg_half(vhalf, attr, ex, src, dst, aw, with_attr):
    body = functools.partial(_sc_agg_body, aw, with_attr)
    return pl.kernel(
        body,
        jax.ShapeDtypeStruct((NC * N, aw), jnp.float32),
        mesh=_mesh(),
        compiler_params=pltpu.CompilerParams(use_tc_tiling_on_sc=False, needs_layout_passes=False),
        scratch_types=[
            pltpu.VMEM((B,), jnp.int32),
            pltpu.VMEM((B,), jnp.int32),
            pltpu.VMEM((B, D // 2), jnp.float32),
            pltpu.VMEM((B, F), jnp.float32),
            pltpu.VMEM((B, H), jnp.float32),
            pltpu.VMEM((B, aw), jnp.float32),
            pltpu.VMEM_SHARED((N, aw), jnp.float32),
            pltpu.SemaphoreType.DMA,
        ],
    )(vhalf, attr, ex, src, dst)


def _sc_agg(va, vb, attr, ex, src, dst):
    acc1 = _sc_agg_half(va, attr, ex, src, dst, AW1, True)
    acc2 = _sc_agg_half(vb, attr, ex, src, dst, AW2, False)
    return acc1, acc2


B2 = 1000


def _sc_anorm_body(ex1, ex2, ex3, ds1, ds2, ds3, dst_hbm, a1, a2, a3,
                   dstb, exb, anb, denb):
    wid = lax.axis_index("c") * NS + lax.axis_index("s")
    for ex_hbm, dsum, a_hbm in ((ex1, ds1, a1), (ex2, ds2, a2), (ex3, ds3, a3)):
        pltpu.sync_copy(dsum, denb)

        def chunk(i, _):
            base = wid * EPW + i * B2
            pltpu.sync_copy(dst_hbm.at[pl.ds(base, B2)], dstb)
            pltpu.sync_copy(ex_hbm.at[pl.ds(base, B2)], exb)

            def group(g, _):
                j = g * 16 + _iota16()
                e = lax.shift_right_logical(j, 2)
                hh = lax.bitwise_and(j, 3)
                dste = plsc.load_gather(dstb, [e])
                denv = plsc.load_gather(denb, [dste, hh])
                exv = plsc.load_gather(exb, [e, hh])
                plsc.store_scatter(anb, [e, hh], exv / (denv + 1e-16))
                return _

            lax.fori_loop(0, B2 * H // 16, group, None)
            pltpu.sync_copy(anb, a_hbm.at[pl.ds(base, B2)])
            return _

        lax.fori_loop(0, EPW // B2, chunk, None)


def _sc_anorm(ex1, ex2, ex3, ds1, ds2, ds3, dst):
    shp = jax.ShapeDtypeStruct((E, H), jnp.float32)
    return pl.kernel(
        _sc_anorm_body,
        (shp, shp, shp),
        mesh=_mesh(),
        compiler_params=pltpu.CompilerParams(use_tc_tiling_on_sc=False, needs_layout_passes=False),
        scratch_types=[
            pltpu.VMEM((B2,), jnp.int32),
            pltpu.VMEM((B2, H), jnp.float32),
            pltpu.VMEM((B2, H), jnp.float32),
            pltpu.VMEM((N, H), jnp.float32),
        ],
    )(ex1, ex2, ex3, ds1, ds2, ds3, dst)


def _proj_body(h_ref, wall_ref, ball_ref, g_ref,
               qq_ref, k_ref, va_ref, vb_ref, s_ref):
    x = h_ref[...]
    qkvs = jnp.dot(x, wall_ref[...], preferred_element_type=jnp.float32)
    qkvs = qkvs + ball_ref[...]
    q = qkvs[:, :D]
    qw = jnp.dot(q, g_ref[...], preferred_element_type=jnp.float32)
    qq_ref[...] = jnp.concatenate([q, qw], axis=1)
    k_ref[...] = qkvs[:, D:2 * D]
    va_ref[...] = qkvs[:, 2 * D:2 * D + D // 2]
    vb_ref[...] = qkvs[:, 2 * D + D // 2:3 * D]
    s_ref[...] = qkvs[:, 3 * D:4 * D]


def _proj(h, wall, ball, gmat):
    return pl.pallas_call(
        _proj_body,
        grid=(GRID,),
        in_specs=[
            pl.BlockSpec((ROWBLK, D), lambda i: (i, 0)),
            pl.BlockSpec((D, 4 * D), lambda i: (0, 0)),
            pl.BlockSpec((1, 4 * D), lambda i: (0, 0)),
            pl.BlockSpec((D, H * F), lambda i: (0, 0)),
        ],
        out_specs=[
            pl.BlockSpec((ROWBLK, QQW), lambda i: (i, 0)),
            pl.BlockSpec((ROWBLK, D), lambda i: (i, 0)),
            pl.BlockSpec((ROWBLK, D // 2), lambda i: (i, 0)),
            pl.BlockSpec((ROWBLK, D // 2), lambda i: (i, 0)),
            pl.BlockSpec((ROWBLK, D), lambda i: (i, 0)),
        ],
        out_shape=[
            jax.ShapeDtypeStruct((N, QQW), jnp.float32),
            jax.ShapeDtypeStruct((N, D), jnp.float32),
            jax.ShapeDtypeStruct((N, D // 2), jnp.float32),
            jax.ShapeDtypeStruct((N, D // 2), jnp.float32),
            jax.ShapeDtypeStruct((N, D), jnp.float32),
        ],
    )(h, wall, ball, gmat)


def _post_body(do_ln, a1t_ref, a1b_ref, a2t_ref, a2b_ref, s_ref,
               weblk_ref, e32_ref, e16_ref, g_ref, b_ref, out_ref, dsum_ref):
    a1 = a1t_ref[...] + a1b_ref[...]
    a2 = a2t_ref[...] + a2b_ref[...]
    den = a2[:, D // 2:D // 2 + H]
    dsum_ref[...] = den
    rec = 1.0 / jnp.where(den > 0.0, den, 1.0)
    rec32 = jnp.dot(rec, e32_ref[...], preferred_element_type=jnp.float32)
    rec16 = jnp.dot(rec, e16_ref[...], preferred_element_type=jnp.float32)
    out_v = jnp.concatenate(
        [a1[:, :D // 2] * rec32[:, :D // 2],
         a2[:, :D // 2] * rec32[:, D // 2:]], axis=1)
    out_e = jnp.dot(a1[:, D // 2:] * rec16, weblk_ref[...],
                    preferred_element_type=jnp.float32)
    out = out_v + out_e + s_ref[...]
    if do_ln:
        mu = jnp.mean(out, axis=1, keepdims=True)
        var = jnp.mean((out - mu) ** 2, axis=1, keepdims=True)
        out = (out - mu) * lax.rsqrt(var + 1e-5) * g_ref[...] + b_ref[...]
        out = jnp.maximum(out, 0.0)
    out_ref[...] = out


def _post(acc1, acc2, s, weblk, e32, e16, lng, lnb, do_ln):
    nb = N // ROWBLK
    return pl.pallas_call(
        functools.partial(_post_body, do_ln),
        grid=(GRID,),
        in_specs=[
            pl.BlockSpec((ROWBLK, AW1), lambda i: (i, 0)),
            pl.BlockSpec((ROWBLK, AW1), lambda i: (nb + i, 0)),
            pl.BlockSpec((ROWBLK, AW2), lambda i: (i, 0)),
            pl.BlockSpec((ROWBLK, AW2), lambda i: (nb + i, 0)),
            pl.BlockSpec((ROWBLK, D), lambda i: (i, 0)),
            pl.BlockSpec((H * F, D), lambda i: (0, 0)),
            pl.BlockSpec((H, D), lambda i: (0, 0)),
            pl.BlockSpec((H, H * F), lambda i: (0, 0)),
            pl.BlockSpec((1, D), lambda i: (0, 0)),
            pl.BlockSpec((1, D), lambda i: (0, 0)),
        ],
        out_specs=[
            pl.BlockSpec((ROWBLK, D), lambda i: (i, 0)),
            pl.BlockSpec((ROWBLK, H), lambda i: (i, 0)),
        ],
        out_shape=[
            jax.ShapeDtypeStruct((N, D), jnp.float32),
            jax.ShapeDtypeStruct((N, H), jnp.float32),
        ],
    )(acc1, acc1, acc2, acc2, s, weblk, e32, e16, lng, lnb)


def _head_body(z_ref, bf_ref, b0_ref, wf1_ref, bf1_ref, lncg_ref, lncb_ref,
               wf2_ref, bf2_ref, outp_ref, sel_ref, zsum, cnt):
    i = pl.program_id(0)

    @pl.when(i == 0)
    def _():
        zsum[...] = jnp.zeros_like(zsum)
        cnt[...] = jnp.zeros_like(cnt)

    bb = bf_ref[...]
    iota_g = lax.broadcasted_iota(jnp.int32, (1, G), 1).astype(jnp.float32)
    onehot = jnp.where(bb == iota_g, 1.0, 0.0)
    zsum[...] += lax.dot_general(onehot, z_ref[...], (((0,), (0,)), ((), ())),
                                 preferred_element_type=jnp.float32)
    cnt[...] += lax.dot_general(onehot, jnp.ones((ROWBLK, 1), jnp.float32),
                                (((0,), (0,)), ((), ())),
                                preferred_element_type=jnp.float32)

    @pl.when(i == GRID - 1)
    def _():
        z_pool = zsum[...] * (1.0 / jnp.maximum(cnt[...], 1.0))
        h1 = jnp.dot(z_pool, wf1_ref[...],
                     preferred_element_type=jnp.float32) + bf1_ref[...]
        mu = jnp.mean(h1, axis=1, keepdims=True)
        var = jnp.mean((h1 - mu) ** 2, axis=1, keepdims=True)
        h1 = (h1 - mu) * lax.rsqrt(var + 1e-5) * lncg_ref[...] + lncb_ref[...]
        h1 = jnp.maximum(h1, 0.0)
        logits = jnp.dot(h1, wf2_ref[...],
                         preferred_element_type=jnp.float32) + bf2_ref[...]
        m = jnp.max(logits, axis=1, keepdims=True)
        ex = jnp.exp(logits - m)
        outp_ref[...] = ex / jnp.sum(ex, axis=1, keepdims=True)
        selhot = jnp.where(
            b0_ref[0, 0] == lax.broadcasted_iota(jnp.int32, (1, G), 1)
            .astype(jnp.float32), 1.0, 0.0)
        sel_ref[...] = jnp.dot(selhot, z_pool,
                               preferred_element_type=jnp.float32)


def _head(z, batchf, b0f, wf1t, bf1, lncg, lncb, wf2t, bf2):
    return pl.pallas_call(
        _head_body,
        grid=(GRID,),
        in_specs=[
            pl.BlockSpec((ROWBLK, D), lambda i: (i, 0)),
            pl.BlockSpec((ROWBLK, 1), lambda i: (i, 0)),
            pl.BlockSpec((1, 1), lambda i: (0, 0)),
            pl.BlockSpec((D, D), lambda i: (0, 0)),
            pl.BlockSpec((1, D), lambda i: (0, 0)),
            pl.BlockSpec((1, D), lambda i: (0, 0)),
            pl.BlockSpec((1, D), lambda i: (0, 0)),
            pl.BlockSpec((D, CLS), lambda i: (0, 0)),
            pl.BlockSpec((1, CLS), lambda i: (0, 0)),
        ],
        out_specs=[
            pl.BlockSpec((G, CLS), lambda i: (0, 0)),
            pl.BlockSpec((1, D), lambda i: (0, 0)),
        ],
        out_shape=[
            jax.ShapeDtypeStruct((G, CLS), jnp.float32),
            jax.ShapeDtypeStruct((1, D), jnp.float32),
        ],
        scratch_shapes=[
            pltpu.VMEM((G, D), jnp.float32),
            pltpu.VMEM((G, 1), jnp.float32),
        ],
    )(z, batchf, b0f, wf1t, bf1, lncg, lncb, wf2t, bf2)


def _pack_layer(p):
    wall = jnp.concatenate([p['Wq'], p['Wk'], p['Wv'], p['Ws']], axis=0).T
    ball = jnp.concatenate([p['bq'], p['bk'], p['bv'], p['bs']])[None, :]
    wr = p['We'].reshape(H, C, F)
    gmat = jax.scipy.linalg.block_diag(*[wr[h] for h in range(H)])
    weblk = jax.scipy.linalg.block_diag(*[wr[h].T for h in range(H)])
    return wall, ball, gmat, weblk


def kernel(x, edge_index, attr, batch, params):
    src = edge_index[0].astype(jnp.int32)
    dst = edge_index[1].astype(jnp.int32)
    e32 = jnp.kron(jnp.eye(H, dtype=jnp.float32), jnp.ones((1, C), jnp.float32))
    e16 = jnp.kron(jnp.eye(H, dtype=jnp.float32), jnp.ones((1, F), jnp.float32))

    h = x
    exs, dsums = [], []
    lns = [(params['ln1_g'], params['ln1_b']), (params['ln2_g'], params['ln2_b']),
           (None, None)]
    for li, cp in enumerate((params['c1'], params['c2'], params['c3'])):
        wall, ball, gmat, weblk = _pack_layer(cp)
        qq, k, va, vb, s = _proj(h, wall, ball, gmat)
        ex = _sc_alpha(qq, k, attr, src, dst)
        acc1, acc2 = _sc_agg(va, vb, attr, ex, src, dst)
        lng, lnb = lns[li]
        do_ln = li < 2
        if not do_ln:
            lng = jnp.ones((D,), jnp.float32)
            lnb = jnp.zeros((D,), jnp.float32)
        h, dsum = _post(acc1, acc2, s, weblk, e32, e16,
                        lng[None, :], lnb[None, :], do_ln)
        exs.append(ex)
        dsums.append(dsum)

    a1, a2, a3 = _sc_anorm(exs[0], exs[1], exs[2],
                           dsums[0], dsums[1], dsums[2], dst)

    z = h
    batchf = batch.astype(jnp.float32)[:, None]
    b0f = batch[0:1].astype(jnp.float32)[:, None]
    out_put, sel = _head(z, batchf, b0f,
                         params['Wf1'].T, params['bf1'][None, :],
                         params['lnc_g'][None, :], params['lnc_b'][None, :],
                         params['Wf2'].T, params['bf2'][None, :])
    return z, sel.reshape(D), out_put, a1, a2, a3


# speedup vs baseline: 9.3666x; 1.2372x over previous
import functools

import jax
import jax.numpy as jnp
from jax import lax
from jax.experimental import pallas as pl
from jax.experimental.pallas import tpu as pltpu
from jax.experimental.pallas import tpu_sc as plsc

H = 4
C = 32
D = H * C
F = 16
N = 10000
E = 320000
G = 16
CLS = 10
QQW = D + H * F
ROWW = D + H * F

NC, NS = 2, 16
NW = NC * NS
EPW = E // NW
B = 80
CH = EPW // B
STRIPE = N // NS
SCALE = 1.0 / (C ** 0.5)

ROWBLK = 2000
GRID = N // ROWBLK

_mesh = lambda: plsc.VectorSubcoreMesh(core_axis_name="c", subcore_axis_name="s")


def _iota16():
    return lax.iota(jnp.int32, 16)


def _full16(v):
    return jnp.full((16,), v, jnp.int32)


def _sc_alpha_body(qq_hbm, k_hbm, attr_hbm, src_hbm, dst_hbm, ex_hbm,
                   srcb0, srcb1, dstb0, dstb1, kr0, kr1, qr0, qr1,
                   ab0, ab1, eb0, eb1,
                   sis0, sis1, sid0, sid1, sk0, sk1, sq0, sq1,
                   sa0, sa1, se0, se1):
    wid = lax.axis_index("c") * NS + lax.axis_index("s")
    srcb, dstb = [srcb0, srcb1], [dstb0, dstb1]
    kr, qr, ab, eb = [kr0, kr1], [qr0, qr1], [ab0, ab1], [eb0, eb1]
    sis, sid = [sis0, sis1], [sid0, sid1]
    sk, sq, sa, se = [sk0, sk1], [sq0, sq1], [sa0, sa1], [se0, se1]

    def issue_idx(i, p):
        base = wid * EPW + i * B
        pltpu.async_copy(src_hbm.at[pl.ds(base, B)], srcb[p], sis[p])
        pltpu.async_copy(dst_hbm.at[pl.ds(base, B)], dstb[p], sid[p])

    def wait_idx(p):
        pltpu.make_async_copy(src_hbm.at[pl.ds(0, B)], srcb[p], sis[p]).wait()
        pltpu.make_async_copy(dst_hbm.at[pl.ds(0, B)], dstb[p], sid[p]).wait()

    def issue_gather(i, p):
        base = wid * EPW + i * B
        pltpu.async_copy(k_hbm.at[srcb[p]], kr[p], sk[p])
        pltpu.async_copy(qq_hbm.at[dstb[p]], qr[p], sq[p])
        pltpu.async_copy(attr_hbm.at[pl.ds(base, B)], ab[p], sa[p])

    def wait_gather(p):
        pltpu.make_async_copy(k_hbm.at[srcb[p]], kr[p], sk[p]).wait()
        pltpu.make_async_copy(qq_hbm.at[dstb[p]], qr[p], sq[p]).wait()
        pltpu.make_async_copy(attr_hbm.at[pl.ds(0, B)], ab[p], sa[p]).wait()

    issue_idx(0, 0)
    issue_idx(1, 1)
    wait_idx(0)
    issue_gather(0, 0)

    def _maybe(cond, fn):
        if isinstance(cond, bool):
            if cond:
                fn()
        else:
            pl.when(cond)(fn)

    def chunk(i, p):
        def _adv():
            wait_idx(1 - p)
            issue_gather(i + 1, 1 - p)
        _maybe(i + 1 < CH, _adv)
        wait_gather(p)
        _maybe(i + 2 < CH, lambda: issue_idx(i + 2, p))
        _maybe(i >= 2, lambda: pltpu.make_async_copy(
            eb[p], ex_hbm.at[pl.ds(0, B)], se[p]).wait())

        krows, qrows, attrb, exb = kr[p], qr[p], ab[p], eb[p]

        def group(g, _):
            elane = g * 16 + _iota16()
            attrv = [plsc.load_gather(attrb, [elane, _full16(f)])
                     for f in range(F)]
            for h in range(H):
                acc = jnp.zeros((16,), jnp.float32)
                for c in range(C):
                    col = _full16(h * C + c)
                    qv = plsc.load_gather(qrows, [elane, col])
                    kv = plsc.load_gather(krows, [elane, col])
                    acc = acc + qv * kv
                for f in range(F):
                    qwv = plsc.load_gather(qrows, [elane, _full16(D + h * F + f)])
                    acc = acc + attrv[f] * qwv
                exv = jnp.exp(acc * SCALE)
                plsc.store_scatter(exb, [elane, _full16(h)], exv)
            return _

        lax.fori_loop(0, B // 16, group, None)
        pltpu.async_copy(exb, ex_hbm.at[pl.ds(wid * EPW + i * B, B)], se[p])

    def pair(t, _):
        chunk(2 * t, 0)
        chunk(2 * t + 1, 1)
        return _

    lax.fori_loop(0, CH // 2, pair, None)
    if CH % 2:
        chunk(CH - 1, 0)
    for p in range(2):
        pltpu.make_async_copy(eb[p], ex_hbm.at[pl.ds(0, B)], se[p]).wait()


def _sc_alpha(qq, k, attr, src, dst):
    return pl.kernel(
        _sc_alpha_body,
        jax.ShapeDtypeStruct((E, H), jnp.float32),
        mesh=_mesh(),
        compiler_params=pltpu.CompilerParams(use_tc_tiling_on_sc=False, needs_layout_passes=False),
        scratch_types=(
            [pltpu.VMEM((B,), jnp.int32)] * 4
            + [pltpu.VMEM((B, D), jnp.float32)] * 2
            + [pltpu.VMEM((B, QQW), jnp.float32)] * 2
            + [pltpu.VMEM((B, F), jnp.float32)] * 2
            + [pltpu.VMEM((B, H), jnp.float32)] * 2
            + [pltpu.SemaphoreType.DMA] * 12
        ),
    )(qq, k, attr, src, dst)


AW1 = 128
AW2 = 80


def _sc_agg_body(aw, with_attr, v_hbm, attr_hbm, ex_hbm, src_hbm, dst_hbm,
                 acc_hbm, srcb0, srcb1, dstb0, dstb1, sd0, sd1,
                 vr0, vr1, ab0, ab1,
                 eb0, eb1, sc0, sc1, acc_s,
                 sis0, sis1, sid0, sid1, sv0, sv1, sa0, sa1, sx0, sx1,
                 ss0, ss1):
    cid = lax.axis_index("c")
    sid = lax.axis_index("s")
    wid = cid * NS + sid
    srcb, dstb, sdst = [srcb0, srcb1], [dstb0, dstb1], [sd0, sd1]
    vr, ab, eb, scb = [vr0, vr1], [ab0, ab1], [eb0, eb1], [sc0, sc1]
    sis, sidm = [sis0, sis1], [sid0, sid1]
    sv, sa, sx, ss = [sv0, sv1], [sa0, sa1], [sx0, sx1], [ss0, ss1]

    def zrow(r, _):
        for kk in range(aw // 16):
            sc0[r, pl.ds(kk * 16, 16)] = jnp.zeros((16,), jnp.float32)
        return _

    lax.fori_loop(0, B, zrow, None)

    soff = sid * STRIPE
    for off in range(0, STRIPE - B + 1, B):
        pltpu.sync_copy(sc0.at[pl.ds(0, B)], acc_s.at[pl.ds(soff + off, B)])
    rem = STRIPE % B
    if rem:
        pltpu.sync_copy(sc0.at[pl.ds(0, rem)],
                        acc_s.at[pl.ds(soff + STRIPE - rem, rem)])
    plsc.subcore_barrier()

    def issue_idx(i, p):
        base = wid * EPW + i * B
        pltpu.async_copy(src_hbm.at[pl.ds(base, B)], srcb[p], sis[p])
        pltpu.async_copy(dst_hbm.at[pl.ds(base, B)], dstb[p], sidm[p])

    def wait_idx(p):
        pltpu.make_async_copy(src_hbm.at[pl.ds(0, B)], srcb[p], sis[p]).wait()
        pltpu.make_async_copy(dst_hbm.at[pl.ds(0, B)], dstb[p], sidm[p]).wait()

    def issue_gather(i, p):
        base = wid * EPW + i * B
        pltpu.async_copy(v_hbm.at[srcb[p]], vr[p], sv[p])
        if with_attr:
            pltpu.async_copy(attr_hbm.at[pl.ds(base, B)], ab[p], sa[p])
        pltpu.async_copy(ex_hbm.at[pl.ds(base, B)], eb[p], sx[p])

    def wait_gather(p):
        pltpu.make_async_copy(v_hbm.at[srcb[p]], vr[p], sv[p]).wait()
        if with_attr:
            pltpu.make_async_copy(attr_hbm.at[pl.ds(0, B)], ab[p], sa[p]).wait()
        pltpu.make_async_copy(ex_hbm.at[pl.ds(0, B)], eb[p], sx[p]).wait()

    issue_idx(0, 0)
    issue_idx(1, 1)
    wait_idx(0)
    issue_gather(0, 0)

    def _maybe(cond, fn):
        if isinstance(cond, bool):
            if cond:
                fn()
        else:
            pl.when(cond)(fn)

    def chunk(i, p):
        def _adv():
            wait_idx(1 - p)
            issue_gather(i + 1, 1 - p)
        _maybe(i + 1 < CH, _adv)
        wait_gather(p)
        _maybe(i >= 2, lambda: pltpu.make_async_copy(
            scb[p], acc_s.at[sdst[p]], ss[p]).wait())
        for kk in range(B // 16):
            sdst[p][pl.ds(kk * 16, 16)] = dstb[p][pl.ds(kk * 16, 16)]
        _maybe(i + 2 < CH, lambda: issue_idx(i + 2, p))

        vrows, attrb, exb, scal = vr[p], ab[p], eb[p], scb[p]

        def group(g, _):
            elane = g * 16 + _iota16()
            exv = [plsc.load_gather(exb, [elane, _full16(h)]) for h in range(H)]
            for c in range(D // 2):
                col = _full16(c)
                vv = plsc.load_gather(vrows, [elane, col])
                plsc.store_scatter(scal, [elane, col],
                                   vv * exv[c // C if aw == AW1 else 2 + c // C])
            if with_attr:
                attrv = [plsc.load_gather(attrb, [elane, _full16(f)])
                         for f in range(F)]
                for h in range(H):
                    for f in range(F):
                        plsc.store_scatter(
                            scal, [elane, _full16(D // 2 + h * F + f)],
                            attrv[f] * exv[h])
            else:
                for h in range(H):
                    plsc.store_scatter(scal, [elane, _full16(D // 2 + h)],
                                       exv[h])
            return _

        lax.fori_loop(0, B // 16, group, None)
        pltpu.async_copy(scal, acc_s.at[sdst[p]], ss[p], add=True)

    def pair(t, _):
        chunk(2 * t, 0)
        chunk(2 * t + 1, 1)
        return _

    lax.fori_loop(0, CH // 2, pair, None)
    if CH % 2:
        chunk(CH - 1, 0)
    for p in range(2):
        pltpu.make_async_copy(scb[p], acc_s.at[sdst[p]], ss[p]).wait()
    plsc.subcore_barrier()

    hoff = cid * N + sid * STRIPE
    pltpu.sync_copy(acc_s.at[pl.ds(soff, STRIPE)], acc_hbm.at[pl.ds(hoff, STRIPE)])


def _sc_agg_half(vhalf, attr, ex, src, dst, aw, with_attr):
    body = functools.partial(_sc_agg_body, aw, with_attr)
    return pl.kernel(
        body,
        jax.ShapeDtypeStruct((NC * N, aw), jnp.float32),
        mesh=_mesh(),
        compiler_params=pltpu.CompilerParams(use_tc_tiling_on_sc=False, needs_layout_passes=False),
        scratch_types=(
            [pltpu.VMEM((B,), jnp.int32)] * 6
            + [pltpu.VMEM((B, D // 2), jnp.float32)] * 2
            + [pltpu.VMEM((B, F), jnp.float32)] * 2
            + [pltpu.VMEM((B, H), jnp.float32)] * 2
            + [pltpu.VMEM((B, aw), jnp.float32)] * 2
            + [pltpu.VMEM_SHARED((N, aw), jnp.float32)]
            + [pltpu.SemaphoreType.DMA] * 12
        ),
    )(vhalf, attr, ex, src, dst)


def _sc_agg(va, vb, attr, ex, src, dst):
    acc1 = _sc_agg_half(va, attr, ex, src, dst, AW1, True)
    acc2 = _sc_agg_half(vb, attr, ex, src, dst, AW2, False)
    return acc1, acc2


B2 = 1000


def _sc_anorm_body(ex1, ex2, ex3, ds1, ds2, ds3, dst_hbm, a1, a2, a3,
                   dstb, exb, anb, denb):
    wid = lax.axis_index("c") * NS + lax.axis_index("s")
    for ex_hbm, dsum, a_hbm in ((ex1, ds1, a1), (ex2, ds2, a2), (ex3, ds3, a3)):
        pltpu.sync_copy(dsum, denb)

        def chunk(i, _):
            base = wid * EPW + i * B2
            pltpu.sync_copy(dst_hbm.at[pl.ds(base, B2)], dstb)
            pltpu.sync_copy(ex_hbm.at[pl.ds(base, B2)], exb)

            def group(g, _):
                j = g * 16 + _iota16()
                e = lax.shift_right_logical(j, 2)
                hh = lax.bitwise_and(j, 3)
                dste = plsc.load_gather(dstb, [e])
                denv = plsc.load_gather(denb, [dste, hh])
                exv = plsc.load_gather(exb, [e, hh])
                plsc.store_scatter(anb, [e, hh], exv / (denv + 1e-16))
                return _

            lax.fori_loop(0, B2 * H // 16, group, None)
            pltpu.sync_copy(anb, a_hbm.at[pl.ds(base, B2)])
            return _

        lax.fori_loop(0, EPW // B2, chunk, None)


def _sc_anorm(ex1, ex2, ex3, ds1, ds2, ds3, dst):
    shp = jax.ShapeDtypeStruct((E, H), jnp.float32)
    return pl.kernel(
        _sc_anorm_body,
        (shp, shp, shp),
        mesh=_mesh(),
        compiler_params=pltpu.CompilerParams(use_tc_tiling_on_sc=False, needs_layout_passes=False),
        scratch_types=[
            pltpu.VMEM((B2,), jnp.int32),
            pltpu.VMEM((B2, H), jnp.float32),
            pltpu.VMEM((B2, H), jnp.float32),
            pltpu.VMEM((N, H), jnp.float32),
        ],
    )(ex1, ex2, ex3, ds1, ds2, ds3, dst)


def _proj_body(h_ref, wall_ref, ball_ref, g_ref,
               qq_ref, k_ref, va_ref, vb_ref, s_ref):
    x = h_ref[...]
    qkvs = jnp.dot(x, wall_ref[...], preferred_element_type=jnp.float32)
    qkvs = qkvs + ball_ref[...]
    q = qkvs[:, :D]
    qw = jnp.dot(q, g_ref[...], preferred_element_type=jnp.float32)
    qq_ref[...] = jnp.concatenate([q, qw], axis=1)
    k_ref[...] = qkvs[:, D:2 * D]
    va_ref[...] = qkvs[:, 2 * D:2 * D + D // 2]
    vb_ref[...] = qkvs[:, 2 * D + D // 2:3 * D]
    s_ref[...] = qkvs[:, 3 * D:4 * D]


def _proj(h, wall, ball, gmat):
    return pl.pallas_call(
        _proj_body,
        grid=(GRID,),
        in_specs=[
            pl.BlockSpec((ROWBLK, D), lambda i: (i, 0)),
            pl.BlockSpec((D, 4 * D), lambda i: (0, 0)),
            pl.BlockSpec((1, 4 * D), lambda i: (0, 0)),
            pl.BlockSpec((D, H * F), lambda i: (0, 0)),
        ],
        out_specs=[
            pl.BlockSpec((ROWBLK, QQW), lambda i: (i, 0)),
            pl.BlockSpec((ROWBLK, D), lambda i: (i, 0)),
            pl.BlockSpec((ROWBLK, D // 2), lambda i: (i, 0)),
            pl.BlockSpec((ROWBLK, D // 2), lambda i: (i, 0)),
            pl.BlockSpec((ROWBLK, D), lambda i: (i, 0)),
        ],
        out_shape=[
            jax.ShapeDtypeStruct((N, QQW), jnp.float32),
            jax.ShapeDtypeStruct((N, D), jnp.float32),
            jax.ShapeDtypeStruct((N, D // 2), jnp.float32),
            jax.ShapeDtypeStruct((N, D // 2), jnp.float32),
            jax.ShapeDtypeStruct((N, D), jnp.float32),
        ],
    )(h, wall, ball, gmat)


def _post_body(do_ln, a1t_ref, a1b_ref, a2t_ref, a2b_ref, s_ref,
               weblk_ref, e32_ref, e16_ref, g_ref, b_ref, out_ref, dsum_ref):
    a1 = a1t_ref[...] + a1b_ref[...]
    a2 = a2t_ref[...] + a2b_ref[...]
    den = a2[:, D // 2:D // 2 + H]
    dsum_ref[...] = den
    rec = 1.0 / jnp.where(den > 0.0, den, 1.0)
    rec32 = jnp.dot(rec, e32_ref[...], preferred_element_type=jnp.float32)
    rec16 = jnp.dot(rec, e16_ref[...], preferred_element_type=jnp.float32)
    out_v = jnp.concatenate(
        [a1[:, :D // 2] * rec32[:, :D // 2],
         a2[:, :D // 2] * rec32[:, D // 2:]], axis=1)
    out_e = jnp.dot(a1[:, D // 2:] * rec16, weblk_ref[...],
                    preferred_element_type=jnp.float32)
    out = out_v + out_e + s_ref[...]
    if do_ln:
        mu = jnp.mean(out, axis=1, keepdims=True)
        var = jnp.mean((out - mu) ** 2, axis=1, keepdims=True)
        out = (out - mu) * lax.rsqrt(var + 1e-5) * g_ref[...] + b_ref[...]
        out = jnp.maximum(out, 0.0)
    out_ref[...] = out


def _post(acc1, acc2, s, weblk, e32, e16, lng, lnb, do_ln):
    nb = N // ROWBLK
    return pl.pallas_call(
        functools.partial(_post_body, do_ln),
        grid=(GRID,),
        in_specs=[
            pl.BlockSpec((ROWBLK, AW1), lambda i: (i, 0)),
            pl.BlockSpec((ROWBLK, AW1), lambda i: (nb + i, 0)),
            pl.BlockSpec((ROWBLK, AW2), lambda i: (i, 0)),
            pl.BlockSpec((ROWBLK, AW2), lambda i: (nb + i, 0)),
            pl.BlockSpec((ROWBLK, D), lambda i: (i, 0)),
            pl.BlockSpec((H * F, D), lambda i: (0, 0)),
            pl.BlockSpec((H, D), lambda i: (0, 0)),
            pl.BlockSpec((H, H * F), lambda i: (0, 0)),
            pl.BlockSpec((1, D), lambda i: (0, 0)),
            pl.BlockSpec((1, D), lambda i: (0, 0)),
        ],
        out_specs=[
            pl.BlockSpec((ROWBLK, D), lambda i: (i, 0)),
            pl.BlockSpec((ROWBLK, H), lambda i: (i, 0)),
        ],
        out_shape=[
            jax.ShapeDtypeStruct((N, D), jnp.float32),
            jax.ShapeDtypeStruct((N, H), jnp.float32),
        ],
    )(acc1, acc1, acc2, acc2, s, weblk, e32, e16, lng, lnb)


def _head_body(z_ref, bf_ref, b0_ref, wf1_ref, bf1_ref, lncg_ref, lncb_ref,
               wf2_ref, bf2_ref, outp_ref, sel_ref, zsum, cnt):
    i = pl.program_id(0)

    @pl.when(i == 0)
    def _():
        zsum[...] = jnp.zeros_like(zsum)
        cnt[...] = jnp.zeros_like(cnt)

    bb = bf_ref[...]
    iota_g = lax.broadcasted_iota(jnp.int32, (1, G), 1).astype(jnp.float32)
    onehot = jnp.where(bb == iota_g, 1.0, 0.0)
    zsum[...] += lax.dot_general(onehot, z_ref[...], (((0,), (0,)), ((), ())),
                                 preferred_element_type=jnp.float32)
    cnt[...] += lax.dot_general(onehot, jnp.ones((ROWBLK, 1), jnp.float32),
                                (((0,), (0,)), ((), ())),
                                preferred_element_type=jnp.float32)

    @pl.when(i == GRID - 1)
    def _():
        z_pool = zsum[...] * (1.0 / jnp.maximum(cnt[...], 1.0))
        h1 = jnp.dot(z_pool, wf1_ref[...],
                     preferred_element_type=jnp.float32) + bf1_ref[...]
        mu = jnp.mean(h1, axis=1, keepdims=True)
        var = jnp.mean((h1 - mu) ** 2, axis=1, keepdims=True)
        h1 = (h1 - mu) * lax.rsqrt(var + 1e-5) * lncg_ref[...] + lncb_ref[...]
        h1 = jnp.maximum(h1, 0.0)
        logits = jnp.dot(h1, wf2_ref[...],
                         preferred_element_type=jnp.float32) + bf2_ref[...]
        m = jnp.max(logits, axis=1, keepdims=True)
        ex = jnp.exp(logits - m)
        outp_ref[...] = ex / jnp.sum(ex, axis=1, keepdims=True)
        selhot = jnp.where(
            b0_ref[0, 0] == lax.broadcasted_iota(jnp.int32, (1, G), 1)
            .astype(jnp.float32), 1.0, 0.0)
        sel_ref[...] = jnp.dot(selhot, z_pool,
                               preferred_element_type=jnp.float32)


def _head(z, batchf, b0f, wf1t, bf1, lncg, lncb, wf2t, bf2):
    return pl.pallas_call(
        _head_body,
        grid=(GRID,),
        in_specs=[
            pl.BlockSpec((ROWBLK, D), lambda i: (i, 0)),
            pl.BlockSpec((ROWBLK, 1), lambda i: (i, 0)),
            pl.BlockSpec((1, 1), lambda i: (0, 0)),
            pl.BlockSpec((D, D), lambda i: (0, 0)),
            pl.BlockSpec((1, D), lambda i: (0, 0)),
            pl.BlockSpec((1, D), lambda i: (0, 0)),
            pl.BlockSpec((1, D), lambda i: (0, 0)),
            pl.BlockSpec((D, CLS), lambda i: (0, 0)),
            pl.BlockSpec((1, CLS), lambda i: (0, 0)),
        ],
        out_specs=[
            pl.BlockSpec((G, CLS), lambda i: (0, 0)),
            pl.BlockSpec((1, D), lambda i: (0, 0)),
        ],
        out_shape=[
            jax.ShapeDtypeStruct((G, CLS), jnp.float32),
            jax.ShapeDtypeStruct((1, D), jnp.float32),
        ],
        scratch_shapes=[
            pltpu.VMEM((G, D), jnp.float32),
            pltpu.VMEM((G, 1), jnp.float32),
        ],
    )(z, batchf, b0f, wf1t, bf1, lncg, lncb, wf2t, bf2)


def _pack_layer(p):
    wall = jnp.concatenate([p['Wq'], p['Wk'], p['Wv'], p['Ws']], axis=0).T
    ball = jnp.concatenate([p['bq'], p['bk'], p['bv'], p['bs']])[None, :]
    wr = p['We'].reshape(H, C, F)
    gmat = jax.scipy.linalg.block_diag(*[wr[h] for h in range(H)])
    weblk = jax.scipy.linalg.block_diag(*[wr[h].T for h in range(H)])
    return wall, ball, gmat, weblk


def kernel(x, edge_index, attr, batch, params):
    src = edge_index[0].astype(jnp.int32)
    dst = edge_index[1].astype(jnp.int32)
    e32 = jnp.kron(jnp.eye(H, dtype=jnp.float32), jnp.ones((1, C), jnp.float32))
    e16 = jnp.kron(jnp.eye(H, dtype=jnp.float32), jnp.ones((1, F), jnp.float32))

    h = x
    exs, dsums = [], []
    lns = [(params['ln1_g'], params['ln1_b']), (params['ln2_g'], params['ln2_b']),
           (None, None)]
    for li, cp in enumerate((params['c1'], params['c2'], params['c3'])):
        wall, ball, gmat, weblk = _pack_layer(cp)
        qq, k, va, vb, s = _proj(h, wall, ball, gmat)
        ex = _sc_alpha(qq, k, attr, src, dst)
        acc1, acc2 = _sc_agg(va, vb, attr, ex, src, dst)
        lng, lnb = lns[li]
        do_ln = li < 2
        if not do_ln:
            lng = jnp.ones((D,), jnp.float32)
            lnb = jnp.zeros((D,), jnp.float32)
        h, dsum = _post(acc1, acc2, s, weblk, e32, e16,
                        lng[None, :], lnb[None, :], do_ln)
        exs.append(ex)
        dsums.append(dsum)

    a1, a2, a3 = _sc_anorm(exs[0], exs[1], exs[2],
                           dsums[0], dsums[1], dsums[2], dst)

    z = h
    batchf = batch.astype(jnp.float32)[:, None]
    b0f = batch[0:1].astype(jnp.float32)[:, None]
    out_put, sel = _head(z, batchf, b0f,
                         params['Wf1'].T, params['bf1'][None, :],
                         params['lnc_g'][None, :], params['lnc_b'][None, :],
                         params['Wf2'].T, params['bf2'][None, :])
    return z, sel.reshape(D), out_put, a1, a2, a3
